# Initial kernel scaffold; baseline (speedup 1.0000x reference)
import jax
import jax.numpy as jnp
from jax.experimental import pallas as pl


def kernel(x, edge_index, W1, b1, W2, b2):
    raise NotImplementedError("write your pallas kernel here")


# speedup vs baseline: 24.6318x; 24.6318x over previous
import functools

import jax
import jax.numpy as jnp
from jax import lax
from jax.experimental import pallas as pl
from jax.experimental.pallas import tpu as pltpu
from jax.experimental.pallas import tpu_sc as plsc

N_NODES = 10000
N_EDGES = 320000
D_FEAT = 128
N_HIDDEN = 64
N_CLASSES = 16

NUM_CORES = 2
NUM_SUBCORES = 16
NUM_TILES = NUM_CORES * NUM_SUBCORES
EPT = N_EDGES // NUM_TILES
CHUNK = 80
NCHUNK = EPT // CHUNK
N_PAD = 10240
ROWS_PER_TILE = N_PAD // NUM_SUBCORES
RZ = 128

_MESH = plsc.VectorSubcoreMesh(core_axis_name="c", subcore_axis_name="s")


def _zero_fill(buf, nrows, width):
    z = jnp.zeros((16,), jnp.float32)
    for r in range(nrows):
        for c in range(width // 16):
            buf[r, pl.ds(c * 16, 16)] = z


def _make_push(width):
    @functools.partial(
        pl.kernel,
        out_type=jax.ShapeDtypeStruct((NUM_CORES, N_PAD, width), jnp.float32),
        mesh=_MESH,
        compiler_params=pltpu.CompilerParams(use_tc_tiling_on_sc=False),
        scratch_types=[
            pltpu.VMEM((EPT,), jnp.int32),
            pltpu.VMEM((NCHUNK, CHUNK), jnp.int32),
            pltpu.VMEM((CHUNK, width), jnp.float32),
            pltpu.VMEM((RZ, width), jnp.float32),
            pltpu.VMEM_SHARED((N_PAD, width), jnp.float32),
            pltpu.SemaphoreType.DMA,
        ],
    )
    def push(g_hbm, src_hbm, dst_hbm, out_hbm, srcv, dstv, rows, stage, acc, sem):
        cid = lax.axis_index("c")
        sid = lax.axis_index("s")
        wid = cid * NUM_SUBCORES + sid

        pltpu.sync_copy(src_hbm.at[wid], srcv)
        pltpu.sync_copy(dst_hbm.at[wid], dstv)

        _zero_fill(stage, RZ, width)
        row0 = sid * ROWS_PER_TILE
        for i in range(ROWS_PER_TILE // RZ):
            pltpu.sync_copy(stage, acc.at[pl.ds(row0 + i * RZ, RZ)])
        plsc.subcore_barrier()

        def body(j, carry):
            pltpu.async_copy(g_hbm.at[srcv.at[pl.ds(j * CHUNK, CHUNK)]], rows, sem).wait()
            pltpu.sync_copy(rows, acc.at[dstv.at[j]], add=True)
            return carry

        lax.fori_loop(0, NCHUNK, body, 0)
        plsc.subcore_barrier()

        for i in range(ROWS_PER_TILE // RZ):
            sl = pl.ds(row0 + i * RZ, RZ)
            pltpu.sync_copy(acc.at[sl], stage)
            pltpu.sync_copy(stage, out_hbm.at[cid, sl])

    return push


_push64 = _make_push(N_HIDDEN)
_push16 = _make_push(N_CLASSES)

DEG_W = 16


@functools.partial(
    pl.kernel,
    out_type=jax.ShapeDtypeStruct((NUM_CORES, N_PAD, DEG_W), jnp.float32),
    mesh=_MESH,
    compiler_params=pltpu.CompilerParams(use_tc_tiling_on_sc=False),
    scratch_types=[
        pltpu.VMEM((NCHUNK, CHUNK), jnp.int32),
        pltpu.VMEM((CHUNK, DEG_W), jnp.float32),
        pltpu.VMEM((RZ, DEG_W), jnp.float32),
        pltpu.VMEM_SHARED((N_PAD, DEG_W), jnp.float32),
    ],
)
def _deg_kernel(dst_hbm, out_hbm, dstv, ones_rows, stage, acc):
    cid = lax.axis_index("c")
    sid = lax.axis_index("s")
    wid = cid * NUM_SUBCORES + sid

    pltpu.sync_copy(dst_hbm.at[wid], dstv)

    one = jnp.ones((16,), jnp.float32)
    for r in range(CHUNK):
        ones_rows[r, pl.ds(0, 16)] = one

    _zero_fill(stage, RZ, DEG_W)
    row0 = sid * ROWS_PER_TILE
    for i in range(ROWS_PER_TILE // RZ):
        pltpu.sync_copy(stage, acc.at[pl.ds(row0 + i * RZ, RZ)])
    plsc.subcore_barrier()

    def body(j, carry):
        pltpu.sync_copy(ones_rows, acc.at[dstv.at[j]], add=True)
        return carry

    lax.fori_loop(0, NCHUNK, body, 0)
    plsc.subcore_barrier()

    for i in range(ROWS_PER_TILE // RZ):
        sl = pl.ds(row0 + i * RZ, RZ)
        pltpu.sync_copy(acc.at[sl], stage)
        pltpu.sync_copy(stage, out_hbm.at[cid, sl])


_BM = 1000


def _tc_a_body(p0, p1, x, w1, g1, dinv):
    deg = p0[:, 0:1] + p1[:, 0:1] + 1.0
    d = lax.rsqrt(deg)
    dinv[...] = d
    g1[...] = d * jnp.dot(x[...], w1[...], preferred_element_type=jnp.float32)


def _tc_a(p0, p1, x, w1):
    grid = (N_NODES // _BM,)
    return pl.pallas_call(
        _tc_a_body,
        grid=grid,
        in_specs=[
            pl.BlockSpec((_BM, DEG_W), lambda i: (i, 0)),
            pl.BlockSpec((_BM, DEG_W), lambda i: (i, 0)),
            pl.BlockSpec((_BM, D_FEAT), lambda i: (i, 0)),
            pl.BlockSpec((D_FEAT, N_HIDDEN), lambda i: (0, 0)),
        ],
        out_specs=[
            pl.BlockSpec((_BM, N_HIDDEN), lambda i: (i, 0)),
            pl.BlockSpec((_BM, 1), lambda i: (i, 0)),
        ],
        out_shape=[
            jax.ShapeDtypeStruct((N_NODES, N_HIDDEN), jnp.float32),
            jax.ShapeDtypeStruct((N_NODES, 1), jnp.float32),
        ],
    )(p0, p1, x, w1)


def _tc_b_body(dinv, a0, a1, g1, b1, w2, g2):
    d = dinv[...]
    z1 = jnp.maximum(d * (a0[...] + a1[...] + g1[...]) + b1[...], 0.0)
    g2[...] = d * jnp.dot(z1, w2[...], preferred_element_type=jnp.float32)


def _tc_b(dinv, a0, a1, g1, b1, w2):
    grid = (N_NODES // _BM,)
    return pl.pallas_call(
        _tc_b_body,
        grid=grid,
        in_specs=[
            pl.BlockSpec((_BM, 1), lambda i: (i, 0)),
            pl.BlockSpec((_BM, N_HIDDEN), lambda i: (i, 0)),
            pl.BlockSpec((_BM, N_HIDDEN), lambda i: (i, 0)),
            pl.BlockSpec((_BM, N_HIDDEN), lambda i: (i, 0)),
            pl.BlockSpec((1, N_HIDDEN), lambda i: (0, 0)),
            pl.BlockSpec((N_HIDDEN, N_CLASSES), lambda i: (0, 0)),
        ],
        out_specs=pl.BlockSpec((_BM, N_CLASSES), lambda i: (i, 0)),
        out_shape=jax.ShapeDtypeStruct((N_NODES, N_CLASSES), jnp.float32),
    )(dinv, a0, a1, g1, b1, w2)


def _tc_c_body(dinv, c0, c1, g2, b2, out):
    z = dinv[...] * (c0[...] + c1[...] + g2[...]) + b2[...]
    m = jnp.max(z, axis=1, keepdims=True)
    e = jnp.exp(z - m)
    out[...] = z - m - jnp.log(jnp.sum(e, axis=1, keepdims=True))


def _tc_c(dinv, c0, c1, g2, b2):
    grid = (N_NODES // _BM,)
    return pl.pallas_call(
        _tc_c_body,
        grid=grid,
        in_specs=[
            pl.BlockSpec((_BM, 1), lambda i: (i, 0)),
            pl.BlockSpec((_BM, N_CLASSES), lambda i: (i, 0)),
            pl.BlockSpec((_BM, N_CLASSES), lambda i: (i, 0)),
            pl.BlockSpec((_BM, N_CLASSES), lambda i: (i, 0)),
            pl.BlockSpec((1, N_CLASSES), lambda i: (0, 0)),
        ],
        out_specs=pl.BlockSpec((_BM, N_CLASSES), lambda i: (i, 0)),
        out_shape=jax.ShapeDtypeStruct((N_NODES, N_CLASSES), jnp.float32),
    )(dinv, c0, c1, g2, b2)


def kernel(x, edge_index, W1, b1, W2, b2):
    src = edge_index[0].reshape(NUM_TILES, EPT)
    dst = edge_index[1].reshape(NUM_TILES, NCHUNK, CHUNK)
    b1r = b1.reshape(1, N_HIDDEN)
    b2r = b2.reshape(1, N_CLASSES)

    p = _deg_kernel(dst)
    g1, dinv = _tc_a(p[0, :N_NODES], p[1, :N_NODES], x, W1)
    a = _push64(g1, src, dst)
    g2 = _tc_b(dinv, a[0, :N_NODES], a[1, :N_NODES], g1, b1r, W2)
    c = _push16(g2, src, dst)
    return _tc_c(dinv, c[0, :N_NODES], c[1, :N_NODES], g2, b2r)


# speedup vs baseline: 42.8748x; 1.7406x over previous
import functools

import jax
import jax.numpy as jnp
from jax import lax
from jax.experimental import pallas as pl
from jax.experimental.pallas import tpu as pltpu
from jax.experimental.pallas import tpu_sc as plsc

N_NODES = 10000
N_EDGES = 320000
D_FEAT = 128
N_HIDDEN = 64
N_CLASSES = 16

NUM_CORES = 2
NUM_SUBCORES = 16
NUM_TILES = NUM_CORES * NUM_SUBCORES
EPT = N_EDGES // NUM_TILES
CHUNK = 80
NCHUNK = EPT // CHUNK
N_PAD = 10240
ROWS_PER_TILE = N_PAD // NUM_SUBCORES
RZ = 128

_MESH = plsc.VectorSubcoreMesh(core_axis_name="c", subcore_axis_name="s")


def _zero_fill(buf, nrows, width):
    z = jnp.zeros((16,), jnp.float32)
    for r in range(nrows):
        for c in range(width // 16):
            buf[r, pl.ds(c * 16, 16)] = z


NB = 5


def _make_push(width):
    @functools.partial(
        pl.kernel,
        out_type=jax.ShapeDtypeStruct((NUM_CORES, N_PAD, width), jnp.float32),
        mesh=_MESH,
        compiler_params=pltpu.CompilerParams(use_tc_tiling_on_sc=False),
        scratch_types=[
            pltpu.VMEM((EPT,), jnp.int32),
            pltpu.VMEM((NCHUNK, CHUNK), jnp.int32),
            pltpu.VMEM((NB, CHUNK, width), jnp.float32),
            pltpu.VMEM((RZ, width), jnp.float32),
            pltpu.VMEM_SHARED((N_PAD, width), jnp.float32),
            pltpu.SemaphoreType.DMA((NB,)),
        ],
    )
    def push(g_hbm, src_hbm, dst_hbm, out_hbm, srcv, dstv, rows, stage, acc, sem):
        cid = lax.axis_index("c")
        sid = lax.axis_index("s")
        wid = cid * NUM_SUBCORES + sid

        pltpu.sync_copy(src_hbm.at[wid], srcv)
        pltpu.sync_copy(dst_hbm.at[wid], dstv)

        _zero_fill(stage, RZ, width)
        row0 = sid * ROWS_PER_TILE
        for i in range(ROWS_PER_TILE // RZ):
            pltpu.sync_copy(stage, acc.at[pl.ds(row0 + i * RZ, RZ)])
        plsc.subcore_barrier()

        def gather_desc(j, b):
            return pltpu.make_async_copy(
                g_hbm.at[srcv.at[pl.ds(j * CHUNK, CHUNK)]], rows.at[b], sem.at[b])

        for b in range(NB - 1):
            gather_desc(b, b).start()

        def outer(g, carry):
            for b in range(NB):
                j = g * NB + b
                jn = j + NB - 1
                nxt = (b + NB - 1) % NB

                @pl.when(jn < NCHUNK)
                def _():
                    gather_desc(jn, nxt).start()

                gather_desc(j, b).wait()
                pltpu.sync_copy(rows.at[b], acc.at[dstv.at[j]], add=True)
            return carry

        lax.fori_loop(0, NCHUNK // NB, outer, 0)
        plsc.subcore_barrier()

        for i in range(ROWS_PER_TILE // RZ):
            sl = pl.ds(row0 + i * RZ, RZ)
            pltpu.sync_copy(acc.at[sl], stage)
            pltpu.sync_copy(stage, out_hbm.at[cid, sl])

    return push


_push64 = _make_push(N_HIDDEN)
_push16 = _make_push(N_CLASSES)

DEG_W = 16


@functools.partial(
    pl.kernel,
    out_type=jax.ShapeDtypeStruct((NUM_CORES, N_PAD, DEG_W), jnp.float32),
    mesh=_MESH,
    compiler_params=pltpu.CompilerParams(use_tc_tiling_on_sc=False),
    scratch_types=[
        pltpu.VMEM((NCHUNK, CHUNK), jnp.int32),
        pltpu.VMEM((CHUNK, DEG_W), jnp.float32),
        pltpu.VMEM((RZ, DEG_W), jnp.float32),
        pltpu.VMEM_SHARED((N_PAD, DEG_W), jnp.float32),
    ],
)
def _deg_kernel(dst_hbm, out_hbm, dstv, ones_rows, stage, acc):
    cid = lax.axis_index("c")
    sid = lax.axis_index("s")
    wid = cid * NUM_SUBCORES + sid

    pltpu.sync_copy(dst_hbm.at[wid], dstv)

    one = jnp.ones((16,), jnp.float32)
    for r in range(CHUNK):
        ones_rows[r, pl.ds(0, 16)] = one

    _zero_fill(stage, RZ, DEG_W)
    row0 = sid * ROWS_PER_TILE
    for i in range(ROWS_PER_TILE // RZ):
        pltpu.sync_copy(stage, acc.at[pl.ds(row0 + i * RZ, RZ)])
    plsc.subcore_barrier()

    def body(j, carry):
        pltpu.sync_copy(ones_rows, acc.at[dstv.at[j]], add=True)
        return carry

    lax.fori_loop(0, NCHUNK, body, 0)
    plsc.subcore_barrier()

    for i in range(ROWS_PER_TILE // RZ):
        sl = pl.ds(row0 + i * RZ, RZ)
        pltpu.sync_copy(acc.at[sl], stage)
        pltpu.sync_copy(stage, out_hbm.at[cid, sl])


_BM = 1000


def _tc_a_body(p0, p1, x, w1, g1, dinv):
    deg = p0[:, 0:1] + p1[:, 0:1] + 1.0
    d = lax.rsqrt(deg)
    dinv[...] = d
    g1[...] = d * jnp.dot(x[...], w1[...], preferred_element_type=jnp.float32)


def _tc_a(p0, p1, x, w1):
    grid = (N_NODES // _BM,)
    return pl.pallas_call(
        _tc_a_body,
        grid=grid,
        in_specs=[
            pl.BlockSpec((_BM, DEG_W), lambda i: (i, 0)),
            pl.BlockSpec((_BM, DEG_W), lambda i: (i, 0)),
            pl.BlockSpec((_BM, D_FEAT), lambda i: (i, 0)),
            pl.BlockSpec((D_FEAT, N_HIDDEN), lambda i: (0, 0)),
        ],
        out_specs=[
            pl.BlockSpec((_BM, N_HIDDEN), lambda i: (i, 0)),
            pl.BlockSpec((_BM, 1), lambda i: (i, 0)),
        ],
        out_shape=[
            jax.ShapeDtypeStruct((N_NODES, N_HIDDEN), jnp.float32),
            jax.ShapeDtypeStruct((N_NODES, 1), jnp.float32),
        ],
    )(p0, p1, x, w1)


def _tc_b_body(dinv, a0, a1, g1, b1, w2, g2):
    d = dinv[...]
    z1 = jnp.maximum(d * (a0[...] + a1[...] + g1[...]) + b1[...], 0.0)
    g2[...] = d * jnp.dot(z1, w2[...], preferred_element_type=jnp.float32)


def _tc_b(dinv, a0, a1, g1, b1, w2):
    grid = (N_NODES // _BM,)
    return pl.pallas_call(
        _tc_b_body,
        grid=grid,
        in_specs=[
            pl.BlockSpec((_BM, 1), lambda i: (i, 0)),
            pl.BlockSpec((_BM, N_HIDDEN), lambda i: (i, 0)),
            pl.BlockSpec((_BM, N_HIDDEN), lambda i: (i, 0)),
            pl.BlockSpec((_BM, N_HIDDEN), lambda i: (i, 0)),
            pl.BlockSpec((1, N_HIDDEN), lambda i: (0, 0)),
            pl.BlockSpec((N_HIDDEN, N_CLASSES), lambda i: (0, 0)),
        ],
        out_specs=pl.BlockSpec((_BM, N_CLASSES), lambda i: (i, 0)),
        out_shape=jax.ShapeDtypeStruct((N_NODES, N_CLASSES), jnp.float32),
    )(dinv, a0, a1, g1, b1, w2)


def _tc_c_body(dinv, c0, c1, g2, b2, out):
    z = dinv[...] * (c0[...] + c1[...] + g2[...]) + b2[...]
    m = jnp.max(z, axis=1, keepdims=True)
    e = jnp.exp(z - m)
    out[...] = z - m - jnp.log(jnp.sum(e, axis=1, keepdims=True))


def _tc_c(dinv, c0, c1, g2, b2):
    grid = (N_NODES // _BM,)
    return pl.pallas_call(
        _tc_c_body,
        grid=grid,
        in_specs=[
            pl.BlockSpec((_BM, 1), lambda i: (i, 0)),
            pl.BlockSpec((_BM, N_CLASSES), lambda i: (i, 0)),
            pl.BlockSpec((_BM, N_CLASSES), lambda i: (i, 0)),
            pl.BlockSpec((_BM, N_CLASSES), lambda i: (i, 0)),
            pl.BlockSpec((1, N_CLASSES), lambda i: (0, 0)),
        ],
        out_specs=pl.BlockSpec((_BM, N_CLASSES), lambda i: (i, 0)),
        out_shape=jax.ShapeDtypeStruct((N_NODES, N_CLASSES), jnp.float32),
    )(dinv, c0, c1, g2, b2)


def kernel(x, edge_index, W1, b1, W2, b2):
    src = edge_index[0].reshape(NUM_TILES, EPT)
    dst = edge_index[1].reshape(NUM_TILES, NCHUNK, CHUNK)
    b1r = b1.reshape(1, N_HIDDEN)
    b2r = b2.reshape(1, N_CLASSES)

    p = _deg_kernel(dst)
    g1, dinv = _tc_a(p[0, :N_NODES], p[1, :N_NODES], x, W1)
    a = _push64(g1, src, dst)
    g2 = _tc_b(dinv, a[0, :N_NODES], a[1, :N_NODES], g1, b1r, W2)
    c = _push16(g2, src, dst)
    return _tc_c(dinv, c[0, :N_NODES], c[1, :N_NODES], g2, b2r)
